# baseline (speedup 1.0000x reference)
import functools

import jax
import jax.numpy as jnp
from jax import lax
from jax.experimental import pallas as pl
from jax.experimental.pallas import tpu as pltpu
from jax.experimental.pallas import tpu_sc as plsc

NC = 2
NS = 16
NW = NC * NS
BATCH = 16384
HIDDEN = 64
CHUNK = 128
B_PER_W = BATCH // NW
NCHUNK = B_PER_W // CHUNK


def _sc_body(users_ref, items_ref, uf_hbm, if_hbm, ub_hbm, ib_hbm, out_hbm,
             uidx_v, iidx_v, uf_v, if_v, ub_v, ib_v, out_v, sem):
    wid = lax.axis_index("s") * NC + lax.axis_index("c")
    row0 = wid * NCHUNK

    pltpu.sync_copy(users_ref.at[pl.ds(row0, NCHUNK)], uidx_v)
    pltpu.sync_copy(items_ref.at[pl.ds(row0, NCHUNK)], iidx_v)

    lanes = jax.lax.iota(jnp.int32, 16)
    last_lane = lanes == 15

    for c in range(NCHUNK):
        cp0 = pltpu.async_copy(uf_hbm.at[uidx_v.at[c]], uf_v, sem)
        cp1 = pltpu.async_copy(if_hbm.at[iidx_v.at[c]], if_v, sem)
        cp2 = pltpu.async_copy(ub_hbm.at[uidx_v.at[c]], ub_v, sem)
        cp3 = pltpu.async_copy(ib_hbm.at[iidx_v.at[c]], ib_v, sem)
        cp0.wait()
        cp1.wait()
        cp2.wait()
        cp3.wait()

        def elem(e, _):
            acc = None
            for j in range(HIDDEN // 16):
                s = pl.ds(j * 16, 16)
                t = uf_v[e, s] * if_v[e, s] + (ub_v[e, s] + ib_v[e, s])
                acc = t if acc is None else acc + t
            sums = plsc.cumsum(acc)
            plsc.store_scatter(out_v,
                               [jnp.full((16,), c * CHUNK + e, jnp.int32)],
                               sums, mask=last_lane)
            return 0

        lax.fori_loop(0, CHUNK, elem, 0, unroll=4)

    pltpu.sync_copy(out_v, out_hbm.at[pl.ds(wid * B_PER_W, B_PER_W)])


@functools.partial(jax.jit, static_argnames=())
def kernel(users, items, user_factors, item_factors, user_biases, item_biases):
    mesh = plsc.VectorSubcoreMesh(
        core_axis_name="c", subcore_axis_name="s",
        num_cores=NC, num_subcores=NS)
    f = pl.kernel(
        _sc_body,
        out_type=jax.ShapeDtypeStruct((BATCH,), jnp.float32),
        mesh=mesh,
        compiler_params=pltpu.CompilerParams(needs_layout_passes=False, use_tc_tiling_on_sc=False),
        scratch_types=[
            pltpu.VMEM((NCHUNK, CHUNK), jnp.int32),
            pltpu.VMEM((NCHUNK, CHUNK), jnp.int32),
            pltpu.VMEM((CHUNK, HIDDEN), jnp.float32),
            pltpu.VMEM((CHUNK, HIDDEN), jnp.float32),
            pltpu.VMEM((CHUNK, HIDDEN), jnp.float32),
            pltpu.VMEM((CHUNK, HIDDEN), jnp.float32),
            pltpu.VMEM((B_PER_W,), jnp.float32),
            pltpu.SemaphoreType.DMA,
        ],
    )
    out = f(users.reshape(BATCH // CHUNK, CHUNK),
            items.reshape(BATCH // CHUNK, CHUNK),
            user_factors, item_factors, user_biases, item_biases)
    return out.reshape(BATCH, 1)
